# Initial kernel scaffold; baseline (speedup 1.0000x reference)
import jax
import jax.numpy as jnp
from jax.experimental import pallas as pl


def kernel(x, batch):
    raise NotImplementedError("write your pallas kernel here")


# speedup vs baseline: 8.8765x; 8.8765x over previous
import jax
import jax.numpy as jnp
from jax import lax
from jax.experimental import pallas as pl
from jax.experimental.pallas import tpu as pltpu
from jax.experimental.pallas import tpu_sc as plsc

N = 50000
K = 20
NB = 16
L = 16
NWORKERS = 32
R = 512
C = 8192
NBLK = (N + R - 1) // R
NROWPAD = NBLK * R
NPAD = N + C
GI_MAX = (NBLK + NWORKERS - 1) // NWORKERS

_INF = float("inf")


def _sload(ref, idx):
    return ref[pl.ds(idx, L)][0]


def _broadcast_lane(vec, lane):
    s = lax.squeeze(lax.slice(vec, (lane,), (lane + 1,)), (0,))
    return jnp.full((L,), s, vec.dtype)


def _merge16(d2m, jvec, k0, i0, k1, i1):
    sb, vb = plsc.sort_key_val(d2m, jvec)
    rb = lax.rev(sb, (0,))
    rvb = lax.rev(vb, (0,))
    c1 = k1 <= rb
    lo_k = jnp.where(c1, k1, rb)
    lo_v = jnp.where(c1, i1, rvb)
    ls, lvs = plsc.sort_key_val(lo_k, lo_v)
    rl = lax.rev(ls, (0,))
    rlv = lax.rev(lvs, (0,))
    c2 = k0 <= rl
    m0 = jnp.where(c2, k0, rl)
    m0v = jnp.where(c2, i0, rlv)
    m1 = jnp.where(c2, rl, k0)
    m1v = jnp.where(c2, rlv, i0)
    k0n, i0n = plsc.sort_key_val(m0, m0v)
    k1n, i1n = plsc.sort_key_val(m1, m1v)
    tau = _broadcast_lane(k1n, K - L - 1)
    return k0n, i0n, k1n, i1n, tau


def _sc_knn(xq_hbm, xs_hbm, ys_hbm, zs_hbm, srow_hbm, erow_hbm,
            out_hbm, qbuf, sbuf, ebuf, xb, yb, zb, kbuf, ibuf):
    wid = lax.axis_index("s") * 2 + lax.axis_index("c")
    iota = lax.iota(jnp.int32, L)

    def chunk_body(t, a0):
        lo = a0 + t * C
        hi = lo + C
        pltpu.sync_copy(xs_hbm.at[pl.ds(lo, C)], xb)
        pltpu.sync_copy(ys_hbm.at[pl.ds(lo, C)], yb)
        pltpu.sync_copy(zs_hbm.at[pl.ds(lo, C)], zb)
        first = jnp.equal(t, 0)
        firstv = jnp.full((L,), first)

        def row_body(r, _):
            q = qbuf[pl.ds(3 * r, L)]
            qxv = _broadcast_lane(q, 0)
            qyv = _broadcast_lane(q, 1)
            qzv = _broadcast_lane(q, 2)
            s = _sload(sbuf, r)
            e = _sload(ebuf, r)
            cs = jnp.maximum(s, lo)
            ce = jnp.minimum(e, hi)
            j0 = ((cs - lo) // L) * L
            n16 = jnp.maximum(0, ((ce - lo) - j0 + (L - 1)) // L)
            k0 = jnp.where(firstv, _INF, kbuf[pl.ds(32 * r, L)])
            k1 = jnp.where(firstv, _INF, kbuf[pl.ds(32 * r + L, L)])
            i0 = jnp.where(firstv, 0, ibuf[pl.ds(32 * r, L)])
            i1 = jnp.where(firstv, 0, ibuf[pl.ds(32 * r + L, L)])
            tau = _broadcast_lane(k1, K - L - 1)
            csv = jnp.full((L,), cs)
            cev = jnp.full((L,), ce)

            def cand_body(c, carry):
                k0, i0, k1, i1, tau = carry
                jl = j0 + c * L
                jvec = (lo + jl) + iota
                dx = xb[pl.ds(jl, L)] - qxv
                dy = yb[pl.ds(jl, L)] - qyv
                dz = zb[pl.ds(jl, L)] - qzv
                d2 = dx * dx + dy * dy + dz * dz
                valid = (jvec >= csv) & (jvec < cev)
                d2m = jnp.where(valid, d2, _INF)
                any_pass = jnp.min(d2m) < tau[0]
                return lax.cond(
                    any_pass,
                    lambda op: _merge16(*op[:6]),
                    lambda op: op[2:],
                    (d2m, jvec, k0, i0, k1, i1, tau),
                )

            k0, i0, k1, i1, tau = lax.fori_loop(
                0, n16, cand_body, (k0, i0, k1, i1, tau))
            kbuf[pl.ds(32 * r, L)] = k0
            kbuf[pl.ds(32 * r + L, L)] = k1
            ibuf[pl.ds(32 * r, L)] = i0
            ibuf[pl.ds(32 * r + L, L)] = i1
            return 0

        lax.fori_loop(0, R, row_body, 0)
        return a0

    def block_body(gi, _):
        g = wid + NWORKERS * gi

        @pl.when(g < NBLK)
        def _():
            base = g * R
            pltpu.sync_copy(xq_hbm.at[pl.ds(3 * base, 3 * R)],
                            qbuf.at[pl.ds(0, 3 * R)])
            pltpu.sync_copy(srow_hbm.at[pl.ds(base, R)],
                            sbuf.at[pl.ds(0, R)])
            pltpu.sync_copy(erow_hbm.at[pl.ds(base, R)],
                            ebuf.at[pl.ds(0, R)])
            count = jnp.minimum(R, N - base)
            s0 = _sload(sbuf, 0)
            e_last = _sload(ebuf, count - 1)
            a0 = (s0 // L) * L
            span = e_last - a0
            nch = (span + C - 1) // C
            lax.fori_loop(0, nch, chunk_body, a0)
            pltpu.sync_copy(ibuf.at[pl.ds(0, 32 * R)],
                            out_hbm.at[pl.ds(32 * base, 32 * R)])

        return 0

    lax.fori_loop(0, GI_MAX, block_body, 0)


def kernel(x, batch):
    x = x.astype(jnp.float32)
    batch = batch.astype(jnp.int32)
    xqf = jnp.pad(x.reshape(-1), (0, 3 * (NROWPAD - N)))
    xs = jnp.pad(x[:, 0], (0, NPAD - N))
    ys = jnp.pad(x[:, 1], (0, NPAD - N))
    zs = jnp.pad(x[:, 2], (0, NPAD - N))
    seg = jnp.arange(NB, dtype=jnp.int32)
    starts = jnp.searchsorted(batch, seg, side="left").astype(jnp.int32)
    ends = jnp.searchsorted(batch, seg, side="right").astype(jnp.int32)
    srow = jnp.pad(starts[batch], (0, NROWPAD - N))
    erow = jnp.pad(ends[batch], (0, NROWPAD - N), constant_values=1)

    mesh = plsc.VectorSubcoreMesh(core_axis_name="c", subcore_axis_name="s")
    f = pl.kernel(
        _sc_knn,
        out_type=jax.ShapeDtypeStruct((NROWPAD * 32,), jnp.int32),
        mesh=mesh,
        compiler_params=pltpu.CompilerParams(needs_layout_passes=False),
        scratch_types=[
            pltpu.VMEM((3 * R + L,), jnp.float32),
            pltpu.VMEM((R + L,), jnp.int32),
            pltpu.VMEM((R + L,), jnp.int32),
            pltpu.VMEM((C,), jnp.float32),
            pltpu.VMEM((C,), jnp.float32),
            pltpu.VMEM((C,), jnp.float32),
            pltpu.VMEM((32 * R,), jnp.float32),
            pltpu.VMEM((32 * R,), jnp.int32),
        ],
    )
    out = f(xqf, xs, ys, zs, srow, erow)
    return out.reshape(NROWPAD, 32)[:N, :K]


# speedup vs baseline: 9.9894x; 1.1254x over previous
import jax
import jax.numpy as jnp
from jax import lax
from jax.experimental import pallas as pl
from jax.experimental.pallas import tpu as pltpu
from jax.experimental.pallas import tpu_sc as plsc

N = 50000
K = 20
NB = 16
L = 16
NWORKERS = 32
R = 392
C = 8192
NBLK = (N + R - 1) // R
NROWPAD = NBLK * R
NPAD = N + C
GI_MAX = NBLK // NWORKERS

_INF = float("inf")


def _sload(ref, idx):
    return ref[pl.ds(idx, L)][0]


def _broadcast_lane(vec, lane):
    s = lax.squeeze(lax.slice(vec, (lane,), (lane + 1,)), (0,))
    return jnp.full((L,), s, vec.dtype)


def _merge16(d2m, jvec, k0, i0, k1, i1):
    sb, vb = plsc.sort_key_val(d2m, jvec)
    rb = lax.rev(sb, (0,))
    rvb = lax.rev(vb, (0,))
    c1 = k1 <= rb
    lo_k = jnp.where(c1, k1, rb)
    lo_v = jnp.where(c1, i1, rvb)
    ls, lvs = plsc.sort_key_val(lo_k, lo_v)
    rl = lax.rev(ls, (0,))
    rlv = lax.rev(lvs, (0,))
    c2 = k0 <= rl
    m0 = jnp.where(c2, k0, rl)
    m0v = jnp.where(c2, i0, rlv)
    m1 = jnp.where(c2, rl, k0)
    m1v = jnp.where(c2, rlv, i0)
    k0n, i0n = plsc.sort_key_val(m0, m0v)
    k1n, i1n = plsc.sort_key_val(m1, m1v)
    tau = _broadcast_lane(k1n, K - L - 1)
    return k0n, i0n, k1n, i1n, tau


def _sc_knn(xq_hbm, xs_hbm, ys_hbm, zs_hbm, srow_hbm, erow_hbm,
            out_hbm, qbuf, sbuf, ebuf, xb, yb, zb, kbuf, ibuf, cdbuf, cibuf):
    wid = lax.axis_index("s") * 2 + lax.axis_index("c")
    iota = lax.iota(jnp.int32, L)

    def chunk_body(t, a0):
        lo = a0 + t * C
        hi = lo + C
        pltpu.sync_copy(xs_hbm.at[pl.ds(lo, C)], xb)
        pltpu.sync_copy(ys_hbm.at[pl.ds(lo, C)], yb)
        pltpu.sync_copy(zs_hbm.at[pl.ds(lo, C)], zb)
        first = jnp.equal(t, 0)
        firstv = jnp.full((L,), first)

        def row_body(r, _):
            q = qbuf[pl.ds(3 * r, L)]
            qxv = _broadcast_lane(q, 0)
            qyv = _broadcast_lane(q, 1)
            qzv = _broadcast_lane(q, 2)
            s = _sload(sbuf, r)
            e = _sload(ebuf, r)
            cs = jnp.maximum(s, lo)
            ce = jnp.minimum(e, hi)
            j0 = ((cs - lo) // L) * L
            n16 = jnp.maximum(0, ((ce - lo) - j0 + (L - 1)) // L)
            k0 = jnp.where(firstv, _INF, kbuf[pl.ds(32 * r, L)])
            k1 = jnp.where(firstv, _INF, kbuf[pl.ds(32 * r + L, L)])
            i0 = jnp.where(firstv, 0, ibuf[pl.ds(32 * r, L)])
            i1 = jnp.where(firstv, 0, ibuf[pl.ds(32 * r + L, L)])
            tau = _broadcast_lane(k1, K - L - 1)
            csv = jnp.full((L,), cs)
            cev = jnp.full((L,), ce)

            def cand_body(c, carry):
                k0, i0, k1, i1, tau, nc = carry
                jl = j0 + c * L
                jvec = (lo + jl) + iota
                dx = xb[pl.ds(jl, L)] - qxv
                dy = yb[pl.ds(jl, L)] - qyv
                dz = zb[pl.ds(jl, L)] - qzv
                d2 = dx * dx + dy * dy + dz * dz
                valid = (jvec >= csv) & (jvec < cev)
                d2m = jnp.where(valid, d2, _INF)
                mask = d2m < tau
                cnt = plsc.all_reduce_population_count(mask)[0]
                plsc.store_compressed(cdbuf.at[pl.ds(nc, L)], d2m, mask=mask)
                plsc.store_compressed(cibuf.at[pl.ds(nc, L)], jvec, mask=mask)
                ncn = nc + cnt

                def flush(op):
                    d16 = cdbuf[0:16]
                    i16 = cibuf[0:16]
                    res = _merge16(d16, i16, *op)
                    tmpd = cdbuf[16:32]
                    tmpi = cibuf[16:32]
                    cdbuf[0:16] = tmpd
                    cibuf[0:16] = tmpi
                    return res

                k0, i0, k1, i1, tau = lax.cond(
                    ncn >= L, flush, lambda op: (op[0], op[1], op[2], op[3], tau),
                    (k0, i0, k1, i1))
                nc = jnp.where(ncn >= L, ncn - L, ncn)
                return (k0, i0, k1, i1, tau, nc)

            k0, i0, k1, i1, tau, nc = lax.fori_loop(
                0, n16, cand_body, (k0, i0, k1, i1, tau, jnp.int32(0)))

            def drain(op):
                lm = iota < jnp.full((L,), nc)
                d16 = jnp.where(lm, cdbuf[0:16], _INF)
                i16 = jnp.where(lm, cibuf[0:16], 0)
                return _merge16(d16, i16, *op)

            k0, i0, k1, i1, tau = lax.cond(
                nc > 0, drain, lambda op: (op[0], op[1], op[2], op[3], tau),
                (k0, i0, k1, i1))
            kbuf[pl.ds(32 * r, L)] = k0
            kbuf[pl.ds(32 * r + L, L)] = k1
            ibuf[pl.ds(32 * r, L)] = i0
            ibuf[pl.ds(32 * r + L, L)] = i1
            return 0

        lax.fori_loop(0, R, row_body, 0)
        return a0

    def block_body(gi, _):
        g = wid + NWORKERS * gi
        base = g * R
        pltpu.sync_copy(xq_hbm.at[pl.ds(3 * base, 3 * R)],
                        qbuf.at[pl.ds(0, 3 * R)])
        pltpu.sync_copy(srow_hbm.at[pl.ds(base, R)],
                        sbuf.at[pl.ds(0, R)])
        pltpu.sync_copy(erow_hbm.at[pl.ds(base, R)],
                        ebuf.at[pl.ds(0, R)])
        count = jnp.minimum(R, N - base)
        s0 = _sload(sbuf, 0)
        e_last = _sload(ebuf, count - 1)
        a0 = (s0 // L) * L
        span = e_last - a0
        nch = (span + C - 1) // C
        lax.fori_loop(0, nch, chunk_body, a0)
        pltpu.sync_copy(ibuf.at[pl.ds(0, 32 * R)],
                        out_hbm.at[pl.ds(32 * base, 32 * R)])
        return 0

    lax.fori_loop(0, GI_MAX, block_body, 0)


def kernel(x, batch):
    x = x.astype(jnp.float32)
    batch = batch.astype(jnp.int32)
    xqf = jnp.pad(x.reshape(-1), (0, 3 * (NROWPAD - N)))
    xs = jnp.pad(x[:, 0], (0, NPAD - N))
    ys = jnp.pad(x[:, 1], (0, NPAD - N))
    zs = jnp.pad(x[:, 2], (0, NPAD - N))
    seg = jnp.arange(NB, dtype=jnp.int32)
    starts = jnp.searchsorted(batch, seg, side="left").astype(jnp.int32)
    ends = jnp.searchsorted(batch, seg, side="right").astype(jnp.int32)
    srow = jnp.pad(starts[batch], (0, NROWPAD - N))
    erow = jnp.pad(ends[batch], (0, NROWPAD - N), constant_values=1)

    mesh = plsc.VectorSubcoreMesh(core_axis_name="c", subcore_axis_name="s")
    f = pl.kernel(
        _sc_knn,
        out_type=jax.ShapeDtypeStruct((NROWPAD * 32,), jnp.int32),
        mesh=mesh,
        compiler_params=pltpu.CompilerParams(needs_layout_passes=False),
        scratch_types=[
            pltpu.VMEM((3 * R + L,), jnp.float32),
            pltpu.VMEM((R + L,), jnp.int32),
            pltpu.VMEM((R + L,), jnp.int32),
            pltpu.VMEM((C,), jnp.float32),
            pltpu.VMEM((C,), jnp.float32),
            pltpu.VMEM((C,), jnp.float32),
            pltpu.VMEM((32 * R,), jnp.float32),
            pltpu.VMEM((32 * R,), jnp.int32),
            pltpu.VMEM((3 * L,), jnp.float32),
            pltpu.VMEM((3 * L,), jnp.int32),
        ],
    )
    out = f(xqf, xs, ys, zs, srow, erow)
    return out.reshape(NROWPAD, 32)[:N, :K]
